# baseline (speedup 1.0000x reference)
import functools

import jax
import jax.numpy as jnp
from jax import lax
from jax.experimental import pallas as pl
from jax.experimental.pallas import tpu as pltpu
from jax.experimental.pallas import tpu_sc as plsc

_E = 16
_K = 2
_NC = 2
_NS = 16
_NW = _NC * _NS
_LANES = 16


def _matmul_body(x_ref, w_ref, b_ref, out_ref):
    out_ref[...] = lax.dot_general(
        w_ref[...], x_ref[...],
        dimension_numbers=(((1,), (1,)), ((), ())),
        preferred_element_type=jnp.float32,
    ) + jnp.transpose(b_ref[...])


def _router_logits_t(x2d, W, b2d):
    T, D = x2d.shape
    BT = 1024
    return pl.pallas_call(
        _matmul_body,
        grid=(T // BT,),
        in_specs=[
            pl.BlockSpec((BT, D), lambda i: (i, 0)),
            pl.BlockSpec((_E, D), lambda i: (0, 0)),
            pl.BlockSpec((1, _E), lambda i: (0, 0)),
        ],
        out_specs=pl.BlockSpec((_E, BT), lambda i: (0, i)),
        out_shape=jax.ShapeDtypeStruct((_E, T), jnp.float32),
    )(x2d, W, b2d)


_NCHUNK = 2


def _route_body(nc, lg_hbm, out_hbm, idx_hbm, lg_v, out_v, idx_v,
                in_sems, out_sems):
    tpw = lg_v.shape[1]
    wpb = out_hbm.shape[2] // tpw
    wid = lax.axis_index("s") * nc + lax.axis_index("c")
    t0 = wid * tpw
    bb = wid // wpb
    s0 = (wid % wpb) * tpw
    tpc = tpw // _NCHUNK

    in_copies = [
        pltpu.async_copy(
            lg_hbm.at[:, pl.ds(t0 + k * tpc, tpc)],
            lg_v.at[:, pl.ds(k * tpc, tpc)],
            in_sems[k],
        )
        for k in range(_NCHUNK)
    ]

    neg_inf = jnp.full((_LANES,), -jnp.inf, dtype=jnp.float32)
    zero_f = jnp.zeros((_LANES,), dtype=jnp.float32)
    zero_i = jnp.zeros((_LANES,), dtype=jnp.int32)

    def group(g, _):
        c0 = g * _LANES
        m1 = lg_v[0, pl.ds(c0, _LANES)]
        i1 = zero_i
        m2, i2 = neg_inf, zero_i
        for e in range(1, _E):
            v = lg_v[e, pl.ds(c0, _LANES)]
            gt1 = v > m1
            gt2 = v > m2
            m2 = jnp.where(gt1, m1, jnp.where(gt2, v, m2))
            i2 = jnp.where(gt1, i1, jnp.where(gt2, e, i2))
            m1 = jnp.where(gt1, v, m1)
            i1 = jnp.where(gt1, e, i1)

        p2e = jnp.exp(m2 - m1)
        s = 1.0 + p2e
        p1 = 1.0 / s
        p2 = p2e / s

        for e in range(_E):
            out_v[e, pl.ds(c0, _LANES)] = jnp.where(
                i1 == e, p1, jnp.where(i2 == e, p2, zero_f)
            )
        idx_v[0, pl.ds(c0, _LANES)] = i1
        idx_v[1, pl.ds(c0, _LANES)] = i2
        return None

    gpc = tpc // _LANES
    out_copies = []
    for k in range(_NCHUNK):
        in_copies[k].wait()
        lax.fori_loop(k * gpc, (k + 1) * gpc, group, None)
        out_copies.append(pltpu.async_copy(
            out_v.at[:, pl.ds(k * tpc, tpc)],
            out_hbm.at[bb, :, pl.ds(s0 + k * tpc, tpc)],
            out_sems[k],
        ))
    pltpu.sync_copy(idx_v, idx_hbm.at[bb, :, pl.ds(s0, tpw)])
    for c in out_copies:
        c.wait()


def _route(lg_t, batch, seq, num_cores=_NC):
    tpw = (batch * seq) // (num_cores * _NS)
    mesh = plsc.VectorSubcoreMesh(
        core_axis_name="c", subcore_axis_name="s", num_cores=num_cores
    )
    fn = functools.partial(
        pl.kernel,
        out_type=[
            jax.ShapeDtypeStruct((batch, _E, seq), jnp.float32),
            jax.ShapeDtypeStruct((batch, _K, seq), jnp.int32),
        ],
        mesh=mesh,
        scratch_types=[
            pltpu.VMEM((_E, tpw), jnp.float32),
            pltpu.VMEM((_E, tpw), jnp.float32),
            pltpu.VMEM((_K, tpw), jnp.int32),
            [pltpu.SemaphoreType.DMA] * _NCHUNK,
            [pltpu.SemaphoreType.DMA] * _NCHUNK,
        ],
    )(functools.partial(_route_body, num_cores))
    return fn(lg_t)


@jax.jit
def kernel(x, W, b):
    B, S, D = x.shape
    T = B * S
    x2d = x.reshape(T, D)
    logits_t = _router_logits_t(x2d, W, b.reshape(1, _E))
    out_t, idx_t = _route(logits_t, B, S)
    return out_t.transpose(0, 2, 1), idx_t.transpose(0, 2, 1)


# speedup vs baseline: 1.0174x; 1.0174x over previous
import functools

import jax
import jax.numpy as jnp
from jax import lax
from jax.experimental import pallas as pl
from jax.experimental.pallas import tpu as pltpu
from jax.experimental.pallas import tpu_sc as plsc

_E = 16
_K = 2
_NC = 2
_NS = 16
_NW = _NC * _NS
_LANES = 16


def _matmul_body(x_ref, w_ref, b_ref, out_ref):
    out_ref[...] = lax.dot_general(
        w_ref[...], x_ref[...],
        dimension_numbers=(((1,), (1,)), ((), ())),
        preferred_element_type=jnp.float32,
    ) + jnp.transpose(b_ref[...])


def _router_logits_t(x2d, W, b2d):
    T, D = x2d.shape
    BT = 1024
    return pl.pallas_call(
        _matmul_body,
        grid=(T // BT,),
        in_specs=[
            pl.BlockSpec((BT, D), lambda i: (i, 0)),
            pl.BlockSpec((_E, D), lambda i: (0, 0)),
            pl.BlockSpec((1, _E), lambda i: (0, 0)),
        ],
        out_specs=pl.BlockSpec((_E, BT), lambda i: (0, i)),
        out_shape=jax.ShapeDtypeStruct((_E, T), jnp.float32),
    )(x2d, W, b2d)


def _route_body(nc, lg_hbm, out_hbm, idx_hbm, lg_v, out_v, idx_v):
    tpw = lg_v.shape[1]
    wpb = out_hbm.shape[2] // tpw
    wid = lax.axis_index("s") * nc + lax.axis_index("c")
    t0 = wid * tpw
    bb = wid // wpb
    s0 = (wid % wpb) * tpw

    pltpu.sync_copy(lg_hbm.at[:, pl.ds(t0, tpw)], lg_v)

    neg_inf = jnp.full((_LANES,), -jnp.inf, dtype=jnp.float32)
    zero_f = jnp.zeros((_LANES,), dtype=jnp.float32)
    zero_i = jnp.zeros((_LANES,), dtype=jnp.int32)

    def group(g, _):
        c0 = g * _LANES
        m1 = lg_v[0, pl.ds(c0, _LANES)]
        i1 = zero_i
        m2, i2 = neg_inf, zero_i
        for e in range(1, _E):
            v = lg_v[e, pl.ds(c0, _LANES)]
            gt1 = v > m1
            gt2 = v > m2
            m2 = jnp.where(gt1, m1, jnp.where(gt2, v, m2))
            i2 = jnp.where(gt1, i1, jnp.where(gt2, e, i2))
            m1 = jnp.where(gt1, v, m1)
            i1 = jnp.where(gt1, e, i1)

        p2e = jnp.exp(m2 - m1)
        s = 1.0 + p2e
        p1 = 1.0 / s
        p2 = p2e / s

        for e in range(_E):
            out_v[e, pl.ds(c0, _LANES)] = jnp.where(
                i1 == e, p1, jnp.where(i2 == e, p2, zero_f)
            )
        idx_v[0, pl.ds(c0, _LANES)] = i1
        idx_v[1, pl.ds(c0, _LANES)] = i2
        return None

    lax.fori_loop(0, tpw // _LANES, group, None)

    pltpu.sync_copy(out_v, out_hbm.at[bb, :, pl.ds(s0, tpw)])
    pltpu.sync_copy(idx_v, idx_hbm.at[bb, :, pl.ds(s0, tpw)])


def _route(lg_t, batch, seq, num_cores=_NC):
    tpw = (batch * seq) // (num_cores * _NS)
    mesh = plsc.VectorSubcoreMesh(
        core_axis_name="c", subcore_axis_name="s", num_cores=num_cores
    )
    fn = functools.partial(
        pl.kernel,
        out_type=[
            jax.ShapeDtypeStruct((batch, _E, seq), jnp.float32),
            jax.ShapeDtypeStruct((batch, _K, seq), jnp.int32),
        ],
        mesh=mesh,
        scratch_types=[
            pltpu.VMEM((_E, tpw), jnp.float32),
            pltpu.VMEM((_E, tpw), jnp.float32),
            pltpu.VMEM((_K, tpw), jnp.int32),
        ],
    )(functools.partial(_route_body, num_cores))
    return fn(lg_t)


@jax.jit
def kernel(x, W, b):
    B, S, D = x.shape
    T = B * S
    x2d = x.reshape(T, D)
    logits_t = _router_logits_t(x2d, W, b.reshape(1, _E))
    out_t, idx_t = _route(logits_t, B, S)
    return out_t.transpose(0, 2, 1), idx_t.transpose(0, 2, 1)
